# baseline (speedup 1.0000x reference)
import functools

import jax
import jax.numpy as jnp
from jax import lax
from jax.experimental import pallas as pl
from jax.experimental.pallas import tpu as pltpu
from jax.experimental.pallas import tpu_sc as plsc

N = 10000
E = 320000
G = 64
R = 128
NP = 10240
EP = 327680
NW = 32
C = 128
K = EP // NW // C
EPH = EP // 2
KH = EPH // NW // C
DR = NP // R
BN = 1024
BE = 2048
F32 = jnp.float32


def _ln(x, g, b):
    m = jnp.mean(x, axis=-1, keepdims=True)
    v = jnp.mean((x - m) * (x - m), axis=-1, keepdims=True)
    return (x - m) * jax.lax.rsqrt(v + 1e-5) * g + b


def _dot(a, b):
    return jnp.dot(a, b, preferred_element_type=F32)


def _leaky(x):
    return jnp.where(x >= 0, x, 0.2 * x)


def _dotb(a, b):
    return jnp.dot(a.astype(jnp.bfloat16), b.astype(jnp.bfloat16),
                   preferred_element_type=F32)


def _mlp_body(x_ref, w0, b0, w1, b1, g, beta, o_ref):
    h = jnp.maximum(_dot(x_ref[...], w0[...]) + b0[...], 0.0)
    o_ref[...] = _ln(_dot(h, w1[...]) + b1[...],
                     g[...], beta[...]).astype(o_ref.dtype)


def _mlp_rows(x, p, bm, out_dtype=F32):
    m, d_in = x.shape
    wspec = [
        pl.BlockSpec((d_in, R), lambda i: (0, 0)),
        pl.BlockSpec((1, R), lambda i: (0, 0)),
        pl.BlockSpec((R, R), lambda i: (0, 0)),
        pl.BlockSpec((1, R), lambda i: (0, 0)),
        pl.BlockSpec((1, R), lambda i: (0, 0)),
        pl.BlockSpec((1, R), lambda i: (0, 0)),
    ]
    return pl.pallas_call(
        _mlp_body,
        grid=(m // bm,),
        in_specs=[pl.BlockSpec((bm, d_in), lambda i: (i, 0))] + wspec,
        out_specs=pl.BlockSpec((bm, R), lambda i: (i, 0)),
        out_shape=jax.ShapeDtypeStruct((m, R), out_dtype),
    )(x, p['W0'], p['b0'].reshape(1, R), p['W1'], p['b1'].reshape(1, R),
      p['g'].reshape(1, R), p['beta'].reshape(1, R))


def _nodeA_body(nd0_ref, nd_ref, wsrc, wdst, w1a, srcT_ref, dstT_ref,
                g1_ref):
    ndc = jnp.concatenate([nd0_ref[...], nd_ref[...]], axis=1)
    srcT_ref[...] = _dot(ndc, wsrc[...])
    dstT_ref[...] = _dot(ndc, wdst[...])
    g1_ref[...] = _dot(ndc, w1a[...])


def _nodeA(nd0, nd, wsrc, wdst, w1a):
    return pl.pallas_call(
        _nodeA_body,
        grid=(NP // BN,),
        in_specs=[
            pl.BlockSpec((BN, R), lambda i: (i, 0)),
            pl.BlockSpec((BN, R), lambda i: (i, 0)),
            pl.BlockSpec((2 * R, R), lambda i: (0, 0)),
            pl.BlockSpec((2 * R, R), lambda i: (0, 0)),
            pl.BlockSpec((2 * R, 2 * R), lambda i: (0, 0)),
        ],
        out_specs=[
            pl.BlockSpec((BN, R), lambda i: (i, 0)),
            pl.BlockSpec((BN, R), lambda i: (i, 0)),
            pl.BlockSpec((BN, 2 * R), lambda i: (i, 0)),
        ],
        out_shape=[
            jax.ShapeDtypeStruct((NP, R), F32),
            jax.ShapeDtypeStruct((NP, R), F32),
            jax.ShapeDtypeStruct((NP, 2 * R), F32),
        ],
    )(nd0, nd, wsrc, wdst, w1a)


def _edgeB_body(msg_ref, ef0_ref, ef_ref, di_ref, wefc, b0, w1e, b1, g, beta,
                w2, a2, efn_ref, rows_ref, den_ref):
    @pl.when(pl.program_id(0) == 0)
    def _():
        den_ref[...] = jnp.zeros_like(den_ref)

    efc = jnp.concatenate([ef0_ref[...].astype(jnp.bfloat16),
                           ef_ref[...].astype(jnp.bfloat16)], axis=1)
    msg = msg_ref[...].astype(F32)
    h = jnp.maximum(msg + _dotb(efc, wefc[...]) + b0[...], 0.0)
    efn = _ln(_dotb(h, w1e[...]) + b1[...], g[...], beta[...])
    efn_ref[...] = efn.astype(efn_ref.dtype)
    g2 = _dotb(efn, w2[...])
    t2 = _dot(_leaky(g2), a2[...])
    ex = jnp.exp(t2)
    rows_ref[...] = ex * g2
    di = di_ref[0, 0, :]
    bm = di.shape[0]
    lane = lax.broadcasted_iota(jnp.int32, (bm, R), 1)
    dlocal = jnp.where(lane == (di & 127)[:, None], ex, 0.0)
    ohhi = (lax.broadcasted_iota(jnp.int32, (DR, bm), 0)
            == lax.shift_right_logical(di, 7)[None, :]).astype(F32)
    den_ref[...] += _dot(ohhi, dlocal)


def _edgeB(msg, ef0, ef, di3, wefc, b0, w1e, b1, g, beta, w2, a2,
           ef_out_dtype=F32, h=0, ef_local=False):
    hb = h * (EPH // BE)
    efmap = (lambda i: (i, 0)) if ef_local else (lambda i, hb=hb: (i + hb, 0))
    return pl.pallas_call(
        _edgeB_body,
        grid=(EPH // BE,),
        in_specs=[
            pl.BlockSpec((BE, R), lambda i: (i, 0)),
            pl.BlockSpec((BE, R), lambda i, hb=hb: (i + hb, 0)),
            pl.BlockSpec((BE, R), efmap),
            pl.BlockSpec((1, 1, BE), lambda i, hb=hb: (i + hb, 0, 0)),
            pl.BlockSpec((2 * R, R), lambda i: (0, 0)),
            pl.BlockSpec((1, R), lambda i: (0, 0)),
            pl.BlockSpec((R, R), lambda i: (0, 0)),
            pl.BlockSpec((1, R), lambda i: (0, 0)),
            pl.BlockSpec((1, R), lambda i: (0, 0)),
            pl.BlockSpec((1, R), lambda i: (0, 0)),
            pl.BlockSpec((R, R), lambda i: (0, 0)),
            pl.BlockSpec((R, 1), lambda i: (0, 0)),
        ],
        out_specs=[
            pl.BlockSpec((BE, R), lambda i: (i, 0)),
            pl.BlockSpec((BE, R), lambda i: (i, 0)),
            pl.BlockSpec((DR, R), lambda i: (0, 0)),
        ],
        out_shape=[
            jax.ShapeDtypeStruct((EPH, R), ef_out_dtype),
            jax.ShapeDtypeStruct((EPH, R), F32),
            jax.ShapeDtypeStruct((DR, R), F32),
        ],
    )(msg, ef0, ef, di3, wefc, b0, w1e, b1, g, beta, w2, a2)


def _nodeD_body(p0_ref, p1_ref, p2_ref, p3_ref, dena_ref, denb_ref, g1_ref,
                w0, b0, w1, b1, g, beta, o_ref):
    num = (p0_ref[...] + p1_ref[...]) + (p2_ref[...] + p3_ref[...])
    den = dena_ref[...] + denb_ref[...]
    rec = 1.0 / (den + 1e-16)
    recb = jnp.reshape(
        jax.lax.broadcast_in_dim(rec, (BN // R, R, R), (0, 1)), (BN, R))
    agg = num * recb
    x = jnp.concatenate([g1_ref[...], agg], axis=1)
    h = jnp.maximum(_dot(x, w0[...]) + b0[...], 0.0)
    o_ref[...] = _ln(_dot(h, w1[...]) + b1[...], g[...], beta[...])


def _nodeD(pa, pb, dena, denb, g1, p):
    nb = NP // BN
    br = BN // R
    return pl.pallas_call(
        _nodeD_body,
        grid=(nb,),
        in_specs=[
            pl.BlockSpec((BN, R), lambda i: (i, 0)),
            pl.BlockSpec((BN, R), lambda i, nb=nb: (i + nb, 0)),
            pl.BlockSpec((BN, R), lambda i: (i, 0)),
            pl.BlockSpec((BN, R), lambda i, nb=nb: (i + nb, 0)),
            pl.BlockSpec((br, R), lambda i: (i, 0)),
            pl.BlockSpec((br, R), lambda i: (i, 0)),
            pl.BlockSpec((BN, 2 * R), lambda i: (i, 0)),
            pl.BlockSpec((3 * R, R), lambda i: (0, 0)),
            pl.BlockSpec((1, R), lambda i: (0, 0)),
            pl.BlockSpec((R, R), lambda i: (0, 0)),
            pl.BlockSpec((1, R), lambda i: (0, 0)),
            pl.BlockSpec((1, R), lambda i: (0, 0)),
            pl.BlockSpec((1, R), lambda i: (0, 0)),
        ],
        out_specs=pl.BlockSpec((BN, R), lambda i: (i, 0)),
        out_shape=jax.ShapeDtypeStruct((NP, R), F32),
    )(pa, pa, pb, pb, dena, denb, g1, p['W0'], p['b0'].reshape(1, R),
      p['W1'], p['b1'].reshape(1, R), p['g'].reshape(1, R),
      p['beta'].reshape(1, R))


def _seg_body(ids_ref, x_ref, o_ref):
    @pl.when(pl.program_id(0) == 0)
    def _():
        o_ref[...] = jnp.zeros_like(o_ref)

    ids = ids_ref[0, 0, :]
    bm = ids.shape[0]
    oh = (lax.broadcasted_iota(jnp.int32, (G, bm), 0)
          == ids[None, :]).astype(F32)
    o_ref[...] += _dot(oh, x_ref[...])


def _seg_agg(ids3, x, bm):
    m = x.shape[0]
    return pl.pallas_call(
        _seg_body,
        grid=(m // bm,),
        in_specs=[
            pl.BlockSpec((1, 1, bm), lambda i: (i, 0, 0)),
            pl.BlockSpec((bm, R), lambda i: (i, 0)),
        ],
        out_specs=pl.BlockSpec((G, R), lambda i: (0, 0)),
        out_shape=jax.ShapeDtypeStruct((G, R), F32),
    )(ids3, x)


def _glob_body(gx_ref, na_ref, ea_ref, w0e, b0e, w1e, b1e, ge, be,
               w0u, b0u, w1u, b1u, gu, bu, o_ref):
    h = jnp.maximum(_dot(gx_ref[...], w0e[...]) + b0e[...], 0.0)
    gd = _ln(_dot(h, w1e[...]) + b1e[...], ge[...], be[...])
    u = jnp.concatenate([gd, na_ref[...], ea_ref[...]], axis=1)
    h2 = jnp.maximum(_dot(u, w0u[...]) + b0u[...], 0.0)
    o_ref[...] = _ln(_dot(h2, w1u[...]) + b1u[...], gu[...], bu[...])


def _glob(gx, na, ea, pe, pu):
    d_g = gx.shape[1]
    return pl.pallas_call(
        _glob_body,
        grid=(1,),
        in_specs=[
            pl.BlockSpec((G, d_g), lambda i: (0, 0)),
            pl.BlockSpec((G, R), lambda i: (0, 0)),
            pl.BlockSpec((G, R), lambda i: (0, 0)),
            pl.BlockSpec((d_g, R), lambda i: (0, 0)),
            pl.BlockSpec((1, R), lambda i: (0, 0)),
            pl.BlockSpec((R, R), lambda i: (0, 0)),
            pl.BlockSpec((1, R), lambda i: (0, 0)),
            pl.BlockSpec((1, R), lambda i: (0, 0)),
            pl.BlockSpec((1, R), lambda i: (0, 0)),
            pl.BlockSpec((3 * R, R), lambda i: (0, 0)),
            pl.BlockSpec((1, R), lambda i: (0, 0)),
            pl.BlockSpec((R, R), lambda i: (0, 0)),
            pl.BlockSpec((1, R), lambda i: (0, 0)),
            pl.BlockSpec((1, R), lambda i: (0, 0)),
            pl.BlockSpec((1, R), lambda i: (0, 0)),
        ],
        out_specs=pl.BlockSpec((G, R), lambda i: (0, 0)),
        out_shape=jax.ShapeDtypeStruct((G, R), F32),
    )(gx, na, ea, pe['W0'], pe['b0'].reshape(1, R), pe['W1'],
      pe['b1'].reshape(1, R), pe['g'].reshape(1, R), pe['beta'].reshape(1, R),
      pu['W0'], pu['b0'].reshape(1, R), pu['W1'], pu['b1'].reshape(1, R),
      pu['g'].reshape(1, R), pu['beta'].reshape(1, R))


def _make_gather_body(h):
  def _sc_gather_body(srcT, dstT, si, di, out,
                      isv0, idv0, isv1, idv1, bufA0, bufB0, bufA1, bufB1,
                      bufO0, bufO1, semg0, semg1, semw0, semw1):
    wid = lax.axis_index("s") * 2 + lax.axis_index("c")
    base = wid * (EPH // NW)
    ibase = h * EPH + base
    isv = (isv0, isv1)
    idv = (idv0, idv1)
    bufA = (bufA0, bufA1)
    bufB = (bufB0, bufB1)
    bufO = (bufO0, bufO1)
    semg = (semg0, semg1)
    semw = (semw0, semw1)
    KK = KH // 2

    pltpu.sync_copy(si.at[pl.ds(ibase, C)], isv0)
    pltpu.sync_copy(di.at[pl.ds(ibase, C)], idv0)
    pltpu.async_copy(srcT.at[isv0], bufA0, semg0)
    pltpu.async_copy(dstT.at[idv0], bufB0, semg0)

    def outer(kk, carry):
        for b in (0, 1):
            bp = 1 - b
            k = 2 * kk + b
            e0 = base + k * C
            i1 = ibase + (k + 1) * C

            def prefetch():
                pltpu.sync_copy(si.at[pl.ds(i1, C)], isv[bp])
                pltpu.sync_copy(di.at[pl.ds(i1, C)], idv[bp])
                pltpu.async_copy(srcT.at[isv[bp]], bufA[bp], semg[bp])
                pltpu.async_copy(dstT.at[idv[bp]], bufB[bp], semg[bp])

            if b == 0:
                prefetch()
            else:
                @pl.when(kk < KK - 1)
                def _():
                    prefetch()

            pltpu.make_async_copy(srcT.at[isv[b]], bufA[b], semg[b]).wait()
            pltpu.make_async_copy(dstT.at[idv[b]], bufB[b], semg[b]).wait()

            @pl.when(kk > 0)
            def _():
                pltpu.make_async_copy(bufO[b], out.at[pl.ds(base, C)],
                                      semw[b]).wait()

            def add_row(r, c2):
                for cc in range(R // 16):
                    sl = pl.ds(cc * 16, 16)
                    bufO[b][r, sl] = bufA[b][r, sl] + bufB[b][r, sl]
                return c2

            lax.fori_loop(0, C, add_row, 0)
            pltpu.async_copy(bufO[b], out.at[pl.ds(e0, C)], semw[b])
        return carry

    lax.fori_loop(0, KK, outer, 0)
    pltpu.make_async_copy(bufO0, out.at[pl.ds(base, C)], semw0).wait()
    pltpu.make_async_copy(bufO1, out.at[pl.ds(base, C)], semw1).wait()
  return _sc_gather_body


@functools.cache
def _sc_gather_kernel(h):
    mesh = plsc.VectorSubcoreMesh(core_axis_name="c", subcore_axis_name="s")
    return pl.kernel(
        _make_gather_body(h),
        out_type=jax.ShapeDtypeStruct((EPH, R), F32),
        mesh=mesh,
        scratch_types=[
            pltpu.VMEM((C,), jnp.int32),
            pltpu.VMEM((C,), jnp.int32),
            pltpu.VMEM((C,), jnp.int32),
            pltpu.VMEM((C,), jnp.int32),
            pltpu.VMEM((C, R), F32),
            pltpu.VMEM((C, R), F32),
            pltpu.VMEM((C, R), F32),
            pltpu.VMEM((C, R), F32),
            pltpu.VMEM((C, R), F32),
            pltpu.VMEM((C, R), F32),
            pltpu.SemaphoreType.DMA,
            pltpu.SemaphoreType.DMA,
            pltpu.SemaphoreType.DMA,
            pltpu.SemaphoreType.DMA,
        ],
    )


def _make_scatter_body(h):
  def _sc_scatter_body(rows, di, zeros, out_num, acc, idv0, idv1, buf0, buf1,
                       semr0, semr1, semsc0, semsc1):
    c = lax.axis_index("c")
    s = lax.axis_index("s")
    wid = s * 2 + c
    base = wid * (EPH // NW)
    ibase = h * EPH + base
    rz = NP // 16
    idv = (idv0, idv1)
    buf = (buf0, buf1)
    semr = (semr0, semr1)
    semsc = (semsc0, semsc1)
    KK = KH // 2

    pltpu.sync_copy(zeros.at[pl.ds(s * rz, rz)], acc.at[pl.ds(s * rz, rz)])
    plsc.subcore_barrier()

    pltpu.sync_copy(di.at[pl.ds(ibase, C)], idv0)
    pltpu.async_copy(rows.at[pl.ds(base, C)], buf0, semr0)

    def outer(kk, carry):
        for b in (0, 1):
            bp = 1 - b
            k = 2 * kk + b
            e0 = base + k * C
            e1 = e0 + C
            i1 = ibase + (k + 1) * C

            pltpu.make_async_copy(rows.at[pl.ds(e0, C)], buf[b],
                                  semr[b]).wait()
            pltpu.async_copy(buf[b], acc.at[idv[b]], semsc[b], add=True)

            def wait_prev():
                pltpu.make_async_copy(buf[bp], acc.at[idv[bp]],
                                      semsc[bp]).wait()

            def prefetch():
                pltpu.sync_copy(di.at[pl.ds(i1, C)], idv[bp])
                pltpu.async_copy(rows.at[pl.ds(e1, C)], buf[bp], semr[bp])

            if b == 0:
                @pl.when(kk > 0)
                def _():
                    wait_prev()

                prefetch()
            else:
                wait_prev()

                @pl.when(kk < KK - 1)
                def _():
                    prefetch()
        return carry

    lax.fori_loop(0, KK, outer, 0)
    pltpu.make_async_copy(buf1, acc.at[idv1], semsc1).wait()
    plsc.subcore_barrier()
    pltpu.sync_copy(acc.at[pl.ds(s * rz, rz)],
                    out_num.at[pl.ds(c * NP + s * rz, rz)])
  return _sc_scatter_body


@functools.cache
def _sc_scatter_kernel(h):
    mesh = plsc.VectorSubcoreMesh(core_axis_name="c", subcore_axis_name="s")
    return pl.kernel(
        _make_scatter_body(h),
        out_type=jax.ShapeDtypeStruct((2 * NP, R), F32),
        mesh=mesh,
        scratch_types=[
            pltpu.VMEM_SHARED((NP, R), F32),
            pltpu.VMEM((C,), jnp.int32),
            pltpu.VMEM((C,), jnp.int32),
            pltpu.VMEM((C, R), F32),
            pltpu.VMEM((C, R), F32),
            pltpu.SemaphoreType.DMA,
            pltpu.SemaphoreType.DMA,
            pltpu.SemaphoreType.DMA,
            pltpu.SemaphoreType.DMA,
        ],
    )


def kernel(node_x, edge_attr, globals_x, params, node_batch, edge_index,
           edge_graph_index):
    node_x_p = jnp.pad(node_x, ((0, NP - N), (0, 0)))
    edge_attr_p = jnp.pad(edge_attr, ((0, EP - E), (0, 0)))
    si = jnp.pad(edge_index[0].astype(jnp.int32), (0, EP - E))
    di = jnp.pad(edge_index[1].astype(jnp.int32), (0, EP - E))
    di_s = jnp.pad(edge_index[1].astype(jnp.int32), (0, EP - E),
                   constant_values=N)
    nb3 = jnp.pad(node_batch.astype(jnp.int32), (0, NP - N),
                  constant_values=G).reshape(NP // BN, 1, BN)
    eg3 = jnp.pad(edge_graph_index.astype(jnp.int32), (0, EP - E),
                  constant_values=G).reshape(EP // BE, 1, BE)
    zeros_acc = jnp.zeros((NP, R), F32)
    di3 = di_s.reshape(EP // BE, 1, BE)

    ap = params['attn']
    pe = params['edge_upd']
    w0 = pe['W0']
    wsrc, wdst, wefc = w0[0:2 * R], w0[2 * R:4 * R], w0[4 * R:6 * R]
    a2 = ap['a'][2 * R:].reshape(R, 1)

    nd0 = _mlp_rows(node_x_p, params['node_enc'], BN)
    ef0 = _mlp_rows(edge_attr_p, params['edge_enc'], BE,
                    out_dtype=jnp.bfloat16)

    nd, ef = nd0, ef0
    for r in range(3):
        srcT, dstT, g1 = _nodeA(nd0, nd, wsrc, wdst, ap['W1'])
        eargs = (di3, wefc, pe['b0'].reshape(1, R),
                 pe['W1'], pe['b1'].reshape(1, R),
                 pe['g'].reshape(1, R), pe['beta'].reshape(1, R),
                 ap['W2'], a2)
        edt = F32 if r == 2 else jnp.bfloat16
        msg0 = _sc_gather_kernel(0)(srcT, dstT, si, di)
        msg1 = _sc_gather_kernel(1)(srcT, dstT, si, di)
        ef_a = ef[0] if isinstance(ef, tuple) else ef
        ef_b = ef[1] if isinstance(ef, tuple) else ef
        loc = isinstance(ef, tuple)
        efa, rows0, den0 = _edgeB(msg0, ef0, ef_a, *eargs,
                                  ef_out_dtype=edt, h=0, ef_local=loc)
        p0 = _sc_scatter_kernel(0)(rows0, di_s, zeros_acc)
        efb, rows1, den1 = _edgeB(msg1, ef0, ef_b, *eargs,
                                  ef_out_dtype=edt, h=1, ef_local=loc)
        p1 = _sc_scatter_kernel(1)(rows1, di_s, zeros_acc)
        nd = _nodeD(p0, p1, den0, den1, g1, params['node_upd'])
        ef = (efa, efb)

    ef = jnp.concatenate([efa, efb], axis=0)
    na = _seg_agg(nb3, nd, BN)
    ea = _seg_agg(eg3, ef, BE)
    gd = _glob(globals_x, na, ea, params['glob_enc'], params['glob_upd'])
    return (nd[:N], ef[:E], gd)


# speedup vs baseline: 1.0486x; 1.0486x over previous
import functools

import jax
import jax.numpy as jnp
from jax import lax
from jax.experimental import pallas as pl
from jax.experimental.pallas import tpu as pltpu
from jax.experimental.pallas import tpu_sc as plsc

N = 10000
E = 320000
G = 64
R = 128
NP = 10240
EP = 327680
NW = 32
C = 128
K = EP // NW // C
EPH = EP // 2
KH = EPH // NW // C
DR = NP // R
BN = 1024
BE = 4096
F32 = jnp.float32


def _ln(x, g, b):
    m = jnp.mean(x, axis=-1, keepdims=True)
    v = jnp.mean((x - m) * (x - m), axis=-1, keepdims=True)
    return (x - m) * jax.lax.rsqrt(v + 1e-5) * g + b


def _dot(a, b):
    return jnp.dot(a, b, preferred_element_type=F32)


def _leaky(x):
    return jnp.where(x >= 0, x, 0.2 * x)


def _dotb(a, b):
    return jnp.dot(a.astype(jnp.bfloat16), b.astype(jnp.bfloat16),
                   preferred_element_type=F32)


def _mlp_body(x_ref, w0, b0, w1, b1, g, beta, o_ref):
    h = jnp.maximum(_dot(x_ref[...], w0[...]) + b0[...], 0.0)
    o_ref[...] = _ln(_dot(h, w1[...]) + b1[...],
                     g[...], beta[...]).astype(o_ref.dtype)


def _mlp_rows(x, p, bm, out_dtype=F32):
    m, d_in = x.shape
    wspec = [
        pl.BlockSpec((d_in, R), lambda i: (0, 0)),
        pl.BlockSpec((1, R), lambda i: (0, 0)),
        pl.BlockSpec((R, R), lambda i: (0, 0)),
        pl.BlockSpec((1, R), lambda i: (0, 0)),
        pl.BlockSpec((1, R), lambda i: (0, 0)),
        pl.BlockSpec((1, R), lambda i: (0, 0)),
    ]
    return pl.pallas_call(
        _mlp_body,
        grid=(m // bm,),
        in_specs=[pl.BlockSpec((bm, d_in), lambda i: (i, 0))] + wspec,
        out_specs=pl.BlockSpec((bm, R), lambda i: (i, 0)),
        out_shape=jax.ShapeDtypeStruct((m, R), out_dtype),
    )(x, p['W0'], p['b0'].reshape(1, R), p['W1'], p['b1'].reshape(1, R),
      p['g'].reshape(1, R), p['beta'].reshape(1, R))


def _nodeA_body(nd0_ref, nd_ref, wsrc, wdst, w1a, srcT_ref, dstT_ref,
                g1_ref):
    ndc = jnp.concatenate([nd0_ref[...], nd_ref[...]], axis=1)
    srcT_ref[...] = _dot(ndc, wsrc[...])
    dstT_ref[...] = _dot(ndc, wdst[...])
    g1_ref[...] = _dot(ndc, w1a[...])


def _nodeA(nd0, nd, wsrc, wdst, w1a):
    return pl.pallas_call(
        _nodeA_body,
        grid=(NP // BN,),
        in_specs=[
            pl.BlockSpec((BN, R), lambda i: (i, 0)),
            pl.BlockSpec((BN, R), lambda i: (i, 0)),
            pl.BlockSpec((2 * R, R), lambda i: (0, 0)),
            pl.BlockSpec((2 * R, R), lambda i: (0, 0)),
            pl.BlockSpec((2 * R, 2 * R), lambda i: (0, 0)),
        ],
        out_specs=[
            pl.BlockSpec((BN, R), lambda i: (i, 0)),
            pl.BlockSpec((BN, R), lambda i: (i, 0)),
            pl.BlockSpec((BN, 2 * R), lambda i: (i, 0)),
        ],
        out_shape=[
            jax.ShapeDtypeStruct((NP, R), F32),
            jax.ShapeDtypeStruct((NP, R), F32),
            jax.ShapeDtypeStruct((NP, 2 * R), F32),
        ],
    )(nd0, nd, wsrc, wdst, w1a)


def _edgeB_body(msg_ref, ef0_ref, ef_ref, di_ref, wefc, b0, w1e, b1, g, beta,
                w2, a2, efn_ref, rows_ref, den_ref):
    @pl.when(pl.program_id(0) == 0)
    def _():
        den_ref[...] = jnp.zeros_like(den_ref)

    efc = jnp.concatenate([ef0_ref[...], ef_ref[...]], axis=1).astype(F32)
    msg = msg_ref[...].astype(F32)
    h = jnp.maximum(msg + _dot(efc, wefc[...]) + b0[...], 0.0)
    efn = _ln(_dot(h, w1e[...]) + b1[...], g[...], beta[...])
    efn_ref[...] = efn.astype(efn_ref.dtype)
    g2 = _dot(efn, w2[...])
    t2 = _dot(_leaky(g2), a2[...])
    ex = jnp.exp(t2)
    rows_ref[...] = ex * g2
    di = di_ref[0, 0, :]
    bm = di.shape[0]
    lane = lax.broadcasted_iota(jnp.int32, (bm, R), 1)
    dlocal = jnp.where(lane == (di & 127)[:, None], ex, 0.0)
    ohhi = (lax.broadcasted_iota(jnp.int32, (DR, bm), 0)
            == lax.shift_right_logical(di, 7)[None, :]).astype(F32)
    den_ref[...] += _dot(ohhi, dlocal)


def _edgeB(msg, ef0, ef, di3, wefc, b0, w1e, b1, g, beta, w2, a2,
           ef_out_dtype=F32, h=0, ef_local=False):
    hb = h * (EPH // BE)
    efmap = (lambda i: (i, 0)) if ef_local else (lambda i, hb=hb: (i + hb, 0))
    return pl.pallas_call(
        _edgeB_body,
        grid=(EPH // BE,),
        in_specs=[
            pl.BlockSpec((BE, R), lambda i: (i, 0)),
            pl.BlockSpec((BE, R), lambda i, hb=hb: (i + hb, 0)),
            pl.BlockSpec((BE, R), efmap),
            pl.BlockSpec((1, 1, BE), lambda i, hb=hb: (i + hb, 0, 0)),
            pl.BlockSpec((2 * R, R), lambda i: (0, 0)),
            pl.BlockSpec((1, R), lambda i: (0, 0)),
            pl.BlockSpec((R, R), lambda i: (0, 0)),
            pl.BlockSpec((1, R), lambda i: (0, 0)),
            pl.BlockSpec((1, R), lambda i: (0, 0)),
            pl.BlockSpec((1, R), lambda i: (0, 0)),
            pl.BlockSpec((R, R), lambda i: (0, 0)),
            pl.BlockSpec((R, 1), lambda i: (0, 0)),
        ],
        out_specs=[
            pl.BlockSpec((BE, R), lambda i: (i, 0)),
            pl.BlockSpec((BE, R), lambda i: (i, 0)),
            pl.BlockSpec((DR, R), lambda i: (0, 0)),
        ],
        out_shape=[
            jax.ShapeDtypeStruct((EPH, R), ef_out_dtype),
            jax.ShapeDtypeStruct((EPH, R), F32),
            jax.ShapeDtypeStruct((DR, R), F32),
        ],
    )(msg, ef0, ef, di3, wefc, b0, w1e, b1, g, beta, w2, a2)


def _nodeD_body(p0_ref, p1_ref, p2_ref, p3_ref, dena_ref, denb_ref, g1_ref,
                w0, b0, w1, b1, g, beta, o_ref):
    num = (p0_ref[...] + p1_ref[...]) + (p2_ref[...] + p3_ref[...])
    den = dena_ref[...] + denb_ref[...]
    rec = 1.0 / (den + 1e-16)
    recb = jnp.reshape(
        jax.lax.broadcast_in_dim(rec, (BN // R, R, R), (0, 1)), (BN, R))
    agg = num * recb
    x = jnp.concatenate([g1_ref[...], agg], axis=1)
    h = jnp.maximum(_dot(x, w0[...]) + b0[...], 0.0)
    o_ref[...] = _ln(_dot(h, w1[...]) + b1[...], g[...], beta[...])


def _nodeD(pa, pb, dena, denb, g1, p):
    nb = NP // BN
    br = BN // R
    return pl.pallas_call(
        _nodeD_body,
        grid=(nb,),
        in_specs=[
            pl.BlockSpec((BN, R), lambda i: (i, 0)),
            pl.BlockSpec((BN, R), lambda i, nb=nb: (i + nb, 0)),
            pl.BlockSpec((BN, R), lambda i: (i, 0)),
            pl.BlockSpec((BN, R), lambda i, nb=nb: (i + nb, 0)),
            pl.BlockSpec((br, R), lambda i: (i, 0)),
            pl.BlockSpec((br, R), lambda i: (i, 0)),
            pl.BlockSpec((BN, 2 * R), lambda i: (i, 0)),
            pl.BlockSpec((3 * R, R), lambda i: (0, 0)),
            pl.BlockSpec((1, R), lambda i: (0, 0)),
            pl.BlockSpec((R, R), lambda i: (0, 0)),
            pl.BlockSpec((1, R), lambda i: (0, 0)),
            pl.BlockSpec((1, R), lambda i: (0, 0)),
            pl.BlockSpec((1, R), lambda i: (0, 0)),
        ],
        out_specs=pl.BlockSpec((BN, R), lambda i: (i, 0)),
        out_shape=jax.ShapeDtypeStruct((NP, R), F32),
    )(pa, pa, pb, pb, dena, denb, g1, p['W0'], p['b0'].reshape(1, R),
      p['W1'], p['b1'].reshape(1, R), p['g'].reshape(1, R),
      p['beta'].reshape(1, R))


def _seg_body(ids_ref, x_ref, o_ref):
    @pl.when(pl.program_id(0) == 0)
    def _():
        o_ref[...] = jnp.zeros_like(o_ref)

    ids = ids_ref[0, 0, :]
    bm = ids.shape[0]
    oh = (lax.broadcasted_iota(jnp.int32, (G, bm), 0)
          == ids[None, :]).astype(F32)
    o_ref[...] += _dot(oh, x_ref[...])


def _seg_agg(ids3, x, bm):
    m = x.shape[0]
    return pl.pallas_call(
        _seg_body,
        grid=(m // bm,),
        in_specs=[
            pl.BlockSpec((1, 1, bm), lambda i: (i, 0, 0)),
            pl.BlockSpec((bm, R), lambda i: (i, 0)),
        ],
        out_specs=pl.BlockSpec((G, R), lambda i: (0, 0)),
        out_shape=jax.ShapeDtypeStruct((G, R), F32),
    )(ids3, x)


def _glob_body(gx_ref, na_ref, ea_ref, w0e, b0e, w1e, b1e, ge, be,
               w0u, b0u, w1u, b1u, gu, bu, o_ref):
    h = jnp.maximum(_dot(gx_ref[...], w0e[...]) + b0e[...], 0.0)
    gd = _ln(_dot(h, w1e[...]) + b1e[...], ge[...], be[...])
    u = jnp.concatenate([gd, na_ref[...], ea_ref[...]], axis=1)
    h2 = jnp.maximum(_dot(u, w0u[...]) + b0u[...], 0.0)
    o_ref[...] = _ln(_dot(h2, w1u[...]) + b1u[...], gu[...], bu[...])


def _glob(gx, na, ea, pe, pu):
    d_g = gx.shape[1]
    return pl.pallas_call(
        _glob_body,
        grid=(1,),
        in_specs=[
            pl.BlockSpec((G, d_g), lambda i: (0, 0)),
            pl.BlockSpec((G, R), lambda i: (0, 0)),
            pl.BlockSpec((G, R), lambda i: (0, 0)),
            pl.BlockSpec((d_g, R), lambda i: (0, 0)),
            pl.BlockSpec((1, R), lambda i: (0, 0)),
            pl.BlockSpec((R, R), lambda i: (0, 0)),
            pl.BlockSpec((1, R), lambda i: (0, 0)),
            pl.BlockSpec((1, R), lambda i: (0, 0)),
            pl.BlockSpec((1, R), lambda i: (0, 0)),
            pl.BlockSpec((3 * R, R), lambda i: (0, 0)),
            pl.BlockSpec((1, R), lambda i: (0, 0)),
            pl.BlockSpec((R, R), lambda i: (0, 0)),
            pl.BlockSpec((1, R), lambda i: (0, 0)),
            pl.BlockSpec((1, R), lambda i: (0, 0)),
            pl.BlockSpec((1, R), lambda i: (0, 0)),
        ],
        out_specs=pl.BlockSpec((G, R), lambda i: (0, 0)),
        out_shape=jax.ShapeDtypeStruct((G, R), F32),
    )(gx, na, ea, pe['W0'], pe['b0'].reshape(1, R), pe['W1'],
      pe['b1'].reshape(1, R), pe['g'].reshape(1, R), pe['beta'].reshape(1, R),
      pu['W0'], pu['b0'].reshape(1, R), pu['W1'], pu['b1'].reshape(1, R),
      pu['g'].reshape(1, R), pu['beta'].reshape(1, R))


def _make_gather_body(h):
  def _sc_gather_body(srcT, dstT, si, di, out,
                      isv0, idv0, isv1, idv1, bufA0, bufB0, bufA1, bufB1,
                      bufO0, bufO1, semg0, semg1, semw0, semw1):
    wid = lax.axis_index("s") * 2 + lax.axis_index("c")
    base = wid * (EPH // NW)
    ibase = h * EPH + base
    isv = (isv0, isv1)
    idv = (idv0, idv1)
    bufA = (bufA0, bufA1)
    bufB = (bufB0, bufB1)
    bufO = (bufO0, bufO1)
    semg = (semg0, semg1)
    semw = (semw0, semw1)
    KK = KH // 2

    pltpu.sync_copy(si.at[pl.ds(ibase, C)], isv0)
    pltpu.sync_copy(di.at[pl.ds(ibase, C)], idv0)
    pltpu.async_copy(srcT.at[isv0], bufA0, semg0)
    pltpu.async_copy(dstT.at[idv0], bufB0, semg0)

    def outer(kk, carry):
        for b in (0, 1):
            bp = 1 - b
            k = 2 * kk + b
            e0 = base + k * C
            i1 = ibase + (k + 1) * C

            def prefetch():
                pltpu.sync_copy(si.at[pl.ds(i1, C)], isv[bp])
                pltpu.sync_copy(di.at[pl.ds(i1, C)], idv[bp])
                pltpu.async_copy(srcT.at[isv[bp]], bufA[bp], semg[bp])
                pltpu.async_copy(dstT.at[idv[bp]], bufB[bp], semg[bp])

            if b == 0:
                prefetch()
            else:
                @pl.when(kk < KK - 1)
                def _():
                    prefetch()

            pltpu.make_async_copy(srcT.at[isv[b]], bufA[b], semg[b]).wait()
            pltpu.make_async_copy(dstT.at[idv[b]], bufB[b], semg[b]).wait()

            @pl.when(kk > 0)
            def _():
                pltpu.make_async_copy(bufO[b], out.at[pl.ds(base, C)],
                                      semw[b]).wait()

            def add_row(r, c2):
                for cc in range(R // 16):
                    sl = pl.ds(cc * 16, 16)
                    bufO[b][r, sl] = bufA[b][r, sl] + bufB[b][r, sl]
                return c2

            lax.fori_loop(0, C, add_row, 0)
            pltpu.async_copy(bufO[b], out.at[pl.ds(e0, C)], semw[b])
        return carry

    lax.fori_loop(0, KK, outer, 0)
    pltpu.make_async_copy(bufO0, out.at[pl.ds(base, C)], semw0).wait()
    pltpu.make_async_copy(bufO1, out.at[pl.ds(base, C)], semw1).wait()
  return _sc_gather_body


@functools.cache
def _sc_gather_kernel(h):
    mesh = plsc.VectorSubcoreMesh(core_axis_name="c", subcore_axis_name="s")
    return pl.kernel(
        _make_gather_body(h),
        out_type=jax.ShapeDtypeStruct((EPH, R), F32),
        mesh=mesh,
        scratch_types=[
            pltpu.VMEM((C,), jnp.int32),
            pltpu.VMEM((C,), jnp.int32),
            pltpu.VMEM((C,), jnp.int32),
            pltpu.VMEM((C,), jnp.int32),
            pltpu.VMEM((C, R), F32),
            pltpu.VMEM((C, R), F32),
            pltpu.VMEM((C, R), F32),
            pltpu.VMEM((C, R), F32),
            pltpu.VMEM((C, R), F32),
            pltpu.VMEM((C, R), F32),
            pltpu.SemaphoreType.DMA,
            pltpu.SemaphoreType.DMA,
            pltpu.SemaphoreType.DMA,
            pltpu.SemaphoreType.DMA,
        ],
    )


def _make_scatter_body(h):
  def _sc_scatter_body(rows, di, zeros, out_num, acc, idv0, idv1, buf0, buf1,
                       semr0, semr1, semsc0, semsc1):
    c = lax.axis_index("c")
    s = lax.axis_index("s")
    wid = s * 2 + c
    base = wid * (EPH // NW)
    ibase = h * EPH + base
    rz = NP // 16
    idv = (idv0, idv1)
    buf = (buf0, buf1)
    semr = (semr0, semr1)
    semsc = (semsc0, semsc1)
    KK = KH // 2

    pltpu.sync_copy(zeros.at[pl.ds(s * rz, rz)], acc.at[pl.ds(s * rz, rz)])
    plsc.subcore_barrier()

    pltpu.sync_copy(di.at[pl.ds(ibase, C)], idv0)
    pltpu.async_copy(rows.at[pl.ds(base, C)], buf0, semr0)

    def outer(kk, carry):
        for b in (0, 1):
            bp = 1 - b
            k = 2 * kk + b
            e0 = base + k * C
            e1 = e0 + C
            i1 = ibase + (k + 1) * C

            pltpu.make_async_copy(rows.at[pl.ds(e0, C)], buf[b],
                                  semr[b]).wait()
            pltpu.async_copy(buf[b], acc.at[idv[b]], semsc[b], add=True)

            def wait_prev():
                pltpu.make_async_copy(buf[bp], acc.at[idv[bp]],
                                      semsc[bp]).wait()

            def prefetch():
                pltpu.sync_copy(di.at[pl.ds(i1, C)], idv[bp])
                pltpu.async_copy(rows.at[pl.ds(e1, C)], buf[bp], semr[bp])

            if b == 0:
                @pl.when(kk > 0)
                def _():
                    wait_prev()

                prefetch()
            else:
                wait_prev()

                @pl.when(kk < KK - 1)
                def _():
                    prefetch()
        return carry

    lax.fori_loop(0, KK, outer, 0)
    pltpu.make_async_copy(buf1, acc.at[idv1], semsc1).wait()
    plsc.subcore_barrier()
    pltpu.sync_copy(acc.at[pl.ds(s * rz, rz)],
                    out_num.at[pl.ds(c * NP + s * rz, rz)])
  return _sc_scatter_body


@functools.cache
def _sc_scatter_kernel(h):
    mesh = plsc.VectorSubcoreMesh(core_axis_name="c", subcore_axis_name="s")
    return pl.kernel(
        _make_scatter_body(h),
        out_type=jax.ShapeDtypeStruct((2 * NP, R), F32),
        mesh=mesh,
        scratch_types=[
            pltpu.VMEM_SHARED((NP, R), F32),
            pltpu.VMEM((C,), jnp.int32),
            pltpu.VMEM((C,), jnp.int32),
            pltpu.VMEM((C, R), F32),
            pltpu.VMEM((C, R), F32),
            pltpu.SemaphoreType.DMA,
            pltpu.SemaphoreType.DMA,
            pltpu.SemaphoreType.DMA,
            pltpu.SemaphoreType.DMA,
        ],
    )


def kernel(node_x, edge_attr, globals_x, params, node_batch, edge_index,
           edge_graph_index):
    node_x_p = jnp.pad(node_x, ((0, NP - N), (0, 0)))
    edge_attr_p = jnp.pad(edge_attr, ((0, EP - E), (0, 0)))
    si = jnp.pad(edge_index[0].astype(jnp.int32), (0, EP - E))
    di = jnp.pad(edge_index[1].astype(jnp.int32), (0, EP - E))
    di_s = jnp.pad(edge_index[1].astype(jnp.int32), (0, EP - E),
                   constant_values=N)
    nb3 = jnp.pad(node_batch.astype(jnp.int32), (0, NP - N),
                  constant_values=G).reshape(NP // BN, 1, BN)
    eg3 = jnp.pad(edge_graph_index.astype(jnp.int32), (0, EP - E),
                  constant_values=G).reshape(EP // BE, 1, BE)
    zeros_acc = jnp.zeros((NP, R), F32)
    di3 = di_s.reshape(EP // BE, 1, BE)

    ap = params['attn']
    pe = params['edge_upd']
    w0 = pe['W0']
    wsrc, wdst, wefc = w0[0:2 * R], w0[2 * R:4 * R], w0[4 * R:6 * R]
    a2 = ap['a'][2 * R:].reshape(R, 1)

    nd0 = _mlp_rows(node_x_p, params['node_enc'], BN)
    ef0 = _mlp_rows(edge_attr_p, params['edge_enc'], BE,
                    out_dtype=jnp.bfloat16)

    nd, ef = nd0, ef0
    for r in range(3):
        srcT, dstT, g1 = _nodeA(nd0, nd, wsrc, wdst, ap['W1'])
        eargs = (di3, wefc, pe['b0'].reshape(1, R),
                 pe['W1'], pe['b1'].reshape(1, R),
                 pe['g'].reshape(1, R), pe['beta'].reshape(1, R),
                 ap['W2'], a2)
        edt = F32 if r == 2 else jnp.bfloat16
        msg0 = _sc_gather_kernel(0)(srcT, dstT, si, di)
        msg1 = _sc_gather_kernel(1)(srcT, dstT, si, di)
        ef_a = ef[0] if isinstance(ef, tuple) else ef
        ef_b = ef[1] if isinstance(ef, tuple) else ef
        loc = isinstance(ef, tuple)
        efa, rows0, den0 = _edgeB(msg0, ef0, ef_a, *eargs,
                                  ef_out_dtype=edt, h=0, ef_local=loc)
        p0 = _sc_scatter_kernel(0)(rows0, di_s, zeros_acc)
        efb, rows1, den1 = _edgeB(msg1, ef0, ef_b, *eargs,
                                  ef_out_dtype=edt, h=1, ef_local=loc)
        p1 = _sc_scatter_kernel(1)(rows1, di_s, zeros_acc)
        nd = _nodeD(p0, p1, den0, den1, g1, params['node_upd'])
        ef = (efa, efb)

    ef = jnp.concatenate([efa, efb], axis=0)
    na = _seg_agg(nb3, nd, BN)
    ea = _seg_agg(eg3, ef, BE)
    gd = _glob(globals_x, na, ea, params['glob_enc'], params['glob_upd'])
    return (nd[:N], ef[:E], gd)


# speedup vs baseline: 1.0584x; 1.0093x over previous
import functools

import jax
import jax.numpy as jnp
from jax import lax
from jax.experimental import pallas as pl
from jax.experimental.pallas import tpu as pltpu
from jax.experimental.pallas import tpu_sc as plsc

N = 10000
E = 320000
G = 64
R = 128
NP = 10240
EP = 327680
NW = 32
C = 128
K = EP // NW // C
EPH = EP // 2
KH = EPH // NW // C
DR = NP // R
BN = 1024
BE = 8192
F32 = jnp.float32


def _ln(x, g, b):
    m = jnp.mean(x, axis=-1, keepdims=True)
    v = jnp.mean((x - m) * (x - m), axis=-1, keepdims=True)
    return (x - m) * jax.lax.rsqrt(v + 1e-5) * g + b


def _dot(a, b):
    return jnp.dot(a, b, preferred_element_type=F32)


def _leaky(x):
    return jnp.where(x >= 0, x, 0.2 * x)


def _dotb(a, b):
    return jnp.dot(a.astype(jnp.bfloat16), b.astype(jnp.bfloat16),
                   preferred_element_type=F32)


def _mlp_body(x_ref, w0, b0, w1, b1, g, beta, o_ref):
    h = jnp.maximum(_dot(x_ref[...], w0[...]) + b0[...], 0.0)
    o_ref[...] = _ln(_dot(h, w1[...]) + b1[...],
                     g[...], beta[...]).astype(o_ref.dtype)


def _mlp_rows(x, p, bm, out_dtype=F32):
    m, d_in = x.shape
    wspec = [
        pl.BlockSpec((d_in, R), lambda i: (0, 0)),
        pl.BlockSpec((1, R), lambda i: (0, 0)),
        pl.BlockSpec((R, R), lambda i: (0, 0)),
        pl.BlockSpec((1, R), lambda i: (0, 0)),
        pl.BlockSpec((1, R), lambda i: (0, 0)),
        pl.BlockSpec((1, R), lambda i: (0, 0)),
    ]
    return pl.pallas_call(
        _mlp_body,
        grid=(m // bm,),
        in_specs=[pl.BlockSpec((bm, d_in), lambda i: (i, 0))] + wspec,
        out_specs=pl.BlockSpec((bm, R), lambda i: (i, 0)),
        out_shape=jax.ShapeDtypeStruct((m, R), out_dtype),
    )(x, p['W0'], p['b0'].reshape(1, R), p['W1'], p['b1'].reshape(1, R),
      p['g'].reshape(1, R), p['beta'].reshape(1, R))


def _nodeA_body(nd0_ref, nd_ref, wsrc, wdst, w1a, srcT_ref, dstT_ref,
                g1_ref):
    ndc = jnp.concatenate([nd0_ref[...], nd_ref[...]], axis=1)
    srcT_ref[...] = _dot(ndc, wsrc[...])
    dstT_ref[...] = _dot(ndc, wdst[...])
    g1_ref[...] = _dot(ndc, w1a[...])


def _nodeA(nd0, nd, wsrc, wdst, w1a):
    return pl.pallas_call(
        _nodeA_body,
        grid=(NP // BN,),
        in_specs=[
            pl.BlockSpec((BN, R), lambda i: (i, 0)),
            pl.BlockSpec((BN, R), lambda i: (i, 0)),
            pl.BlockSpec((2 * R, R), lambda i: (0, 0)),
            pl.BlockSpec((2 * R, R), lambda i: (0, 0)),
            pl.BlockSpec((2 * R, 2 * R), lambda i: (0, 0)),
        ],
        out_specs=[
            pl.BlockSpec((BN, R), lambda i: (i, 0)),
            pl.BlockSpec((BN, R), lambda i: (i, 0)),
            pl.BlockSpec((BN, 2 * R), lambda i: (i, 0)),
        ],
        out_shape=[
            jax.ShapeDtypeStruct((NP, R), F32),
            jax.ShapeDtypeStruct((NP, R), F32),
            jax.ShapeDtypeStruct((NP, 2 * R), F32),
        ],
    )(nd0, nd, wsrc, wdst, w1a)


def _edgeB_body(msg_ref, ef0_ref, ef_ref, di_ref, wefc, b0, w1e, b1, g, beta,
                w2, a2, efn_ref, rows_ref, den_ref):
    @pl.when(pl.program_id(0) == 0)
    def _():
        den_ref[...] = jnp.zeros_like(den_ref)

    efc = jnp.concatenate([ef0_ref[...], ef_ref[...]], axis=1).astype(F32)
    msg = msg_ref[...].astype(F32)
    h = jnp.maximum(msg + _dot(efc, wefc[...]) + b0[...], 0.0)
    efn = _ln(_dot(h, w1e[...]) + b1[...], g[...], beta[...])
    efn_ref[...] = efn.astype(efn_ref.dtype)
    g2 = _dot(efn, w2[...])
    t2 = _dot(_leaky(g2), a2[...])
    ex = jnp.exp(t2)
    rows_ref[...] = ex * g2
    di = di_ref[0, 0, :]
    bm = di.shape[0]
    lane = lax.broadcasted_iota(jnp.int32, (bm, R), 1)
    dlocal = jnp.where(lane == (di & 127)[:, None], ex, 0.0)
    ohhi = (lax.broadcasted_iota(jnp.int32, (DR, bm), 0)
            == lax.shift_right_logical(di, 7)[None, :]).astype(F32)
    den_ref[...] += _dot(ohhi, dlocal)


def _edgeB(msg, ef0, ef, di3, wefc, b0, w1e, b1, g, beta, w2, a2,
           ef_out_dtype=F32, h=0, ef_local=False):
    hb = h * (EPH // BE)
    efmap = (lambda i: (i, 0)) if ef_local else (lambda i, hb=hb: (i + hb, 0))
    return pl.pallas_call(
        _edgeB_body,
        grid=(EPH // BE,),
        in_specs=[
            pl.BlockSpec((BE, R), lambda i: (i, 0)),
            pl.BlockSpec((BE, R), lambda i, hb=hb: (i + hb, 0)),
            pl.BlockSpec((BE, R), efmap),
            pl.BlockSpec((1, 1, BE), lambda i, hb=hb: (i + hb, 0, 0)),
            pl.BlockSpec((2 * R, R), lambda i: (0, 0)),
            pl.BlockSpec((1, R), lambda i: (0, 0)),
            pl.BlockSpec((R, R), lambda i: (0, 0)),
            pl.BlockSpec((1, R), lambda i: (0, 0)),
            pl.BlockSpec((1, R), lambda i: (0, 0)),
            pl.BlockSpec((1, R), lambda i: (0, 0)),
            pl.BlockSpec((R, R), lambda i: (0, 0)),
            pl.BlockSpec((R, 1), lambda i: (0, 0)),
        ],
        out_specs=[
            pl.BlockSpec((BE, R), lambda i: (i, 0)),
            pl.BlockSpec((BE, R), lambda i: (i, 0)),
            pl.BlockSpec((DR, R), lambda i: (0, 0)),
        ],
        out_shape=[
            jax.ShapeDtypeStruct((EPH, R), ef_out_dtype),
            jax.ShapeDtypeStruct((EPH, R), F32),
            jax.ShapeDtypeStruct((DR, R), F32),
        ],
    )(msg, ef0, ef, di3, wefc, b0, w1e, b1, g, beta, w2, a2)


def _nodeD_body(p0_ref, p1_ref, p2_ref, p3_ref, dena_ref, denb_ref, g1_ref,
                w0, b0, w1, b1, g, beta, o_ref):
    num = (p0_ref[...] + p1_ref[...]) + (p2_ref[...] + p3_ref[...])
    den = dena_ref[...] + denb_ref[...]
    rec = 1.0 / (den + 1e-16)
    recb = jnp.reshape(
        jax.lax.broadcast_in_dim(rec, (BN // R, R, R), (0, 1)), (BN, R))
    agg = num * recb
    x = jnp.concatenate([g1_ref[...], agg], axis=1)
    h = jnp.maximum(_dot(x, w0[...]) + b0[...], 0.0)
    o_ref[...] = _ln(_dot(h, w1[...]) + b1[...], g[...], beta[...])


def _nodeD(pa, pb, dena, denb, g1, p):
    nb = NP // BN
    br = BN // R
    return pl.pallas_call(
        _nodeD_body,
        grid=(nb,),
        in_specs=[
            pl.BlockSpec((BN, R), lambda i: (i, 0)),
            pl.BlockSpec((BN, R), lambda i, nb=nb: (i + nb, 0)),
            pl.BlockSpec((BN, R), lambda i: (i, 0)),
            pl.BlockSpec((BN, R), lambda i, nb=nb: (i + nb, 0)),
            pl.BlockSpec((br, R), lambda i: (i, 0)),
            pl.BlockSpec((br, R), lambda i: (i, 0)),
            pl.BlockSpec((BN, 2 * R), lambda i: (i, 0)),
            pl.BlockSpec((3 * R, R), lambda i: (0, 0)),
            pl.BlockSpec((1, R), lambda i: (0, 0)),
            pl.BlockSpec((R, R), lambda i: (0, 0)),
            pl.BlockSpec((1, R), lambda i: (0, 0)),
            pl.BlockSpec((1, R), lambda i: (0, 0)),
            pl.BlockSpec((1, R), lambda i: (0, 0)),
        ],
        out_specs=pl.BlockSpec((BN, R), lambda i: (i, 0)),
        out_shape=jax.ShapeDtypeStruct((NP, R), F32),
    )(pa, pa, pb, pb, dena, denb, g1, p['W0'], p['b0'].reshape(1, R),
      p['W1'], p['b1'].reshape(1, R), p['g'].reshape(1, R),
      p['beta'].reshape(1, R))


def _seg_body(ids_ref, x_ref, o_ref):
    @pl.when(pl.program_id(0) == 0)
    def _():
        o_ref[...] = jnp.zeros_like(o_ref)

    ids = ids_ref[0, 0, :]
    bm = ids.shape[0]
    oh = (lax.broadcasted_iota(jnp.int32, (G, bm), 0)
          == ids[None, :]).astype(F32)
    o_ref[...] += _dot(oh, x_ref[...])


def _seg_agg(ids3, x, bm):
    m = x.shape[0]
    return pl.pallas_call(
        _seg_body,
        grid=(m // bm,),
        in_specs=[
            pl.BlockSpec((1, 1, bm), lambda i: (i, 0, 0)),
            pl.BlockSpec((bm, R), lambda i: (i, 0)),
        ],
        out_specs=pl.BlockSpec((G, R), lambda i: (0, 0)),
        out_shape=jax.ShapeDtypeStruct((G, R), F32),
    )(ids3, x)


def _glob_body(gx_ref, na_ref, ea_ref, w0e, b0e, w1e, b1e, ge, be,
               w0u, b0u, w1u, b1u, gu, bu, o_ref):
    h = jnp.maximum(_dot(gx_ref[...], w0e[...]) + b0e[...], 0.0)
    gd = _ln(_dot(h, w1e[...]) + b1e[...], ge[...], be[...])
    u = jnp.concatenate([gd, na_ref[...], ea_ref[...]], axis=1)
    h2 = jnp.maximum(_dot(u, w0u[...]) + b0u[...], 0.0)
    o_ref[...] = _ln(_dot(h2, w1u[...]) + b1u[...], gu[...], bu[...])


def _glob(gx, na, ea, pe, pu):
    d_g = gx.shape[1]
    return pl.pallas_call(
        _glob_body,
        grid=(1,),
        in_specs=[
            pl.BlockSpec((G, d_g), lambda i: (0, 0)),
            pl.BlockSpec((G, R), lambda i: (0, 0)),
            pl.BlockSpec((G, R), lambda i: (0, 0)),
            pl.BlockSpec((d_g, R), lambda i: (0, 0)),
            pl.BlockSpec((1, R), lambda i: (0, 0)),
            pl.BlockSpec((R, R), lambda i: (0, 0)),
            pl.BlockSpec((1, R), lambda i: (0, 0)),
            pl.BlockSpec((1, R), lambda i: (0, 0)),
            pl.BlockSpec((1, R), lambda i: (0, 0)),
            pl.BlockSpec((3 * R, R), lambda i: (0, 0)),
            pl.BlockSpec((1, R), lambda i: (0, 0)),
            pl.BlockSpec((R, R), lambda i: (0, 0)),
            pl.BlockSpec((1, R), lambda i: (0, 0)),
            pl.BlockSpec((1, R), lambda i: (0, 0)),
            pl.BlockSpec((1, R), lambda i: (0, 0)),
        ],
        out_specs=pl.BlockSpec((G, R), lambda i: (0, 0)),
        out_shape=jax.ShapeDtypeStruct((G, R), F32),
    )(gx, na, ea, pe['W0'], pe['b0'].reshape(1, R), pe['W1'],
      pe['b1'].reshape(1, R), pe['g'].reshape(1, R), pe['beta'].reshape(1, R),
      pu['W0'], pu['b0'].reshape(1, R), pu['W1'], pu['b1'].reshape(1, R),
      pu['g'].reshape(1, R), pu['beta'].reshape(1, R))


def _make_gather_body(h):
  def _sc_gather_body(srcT, dstT, si, di, out,
                      isv0, idv0, isv1, idv1, bufA0, bufB0, bufA1, bufB1,
                      bufO0, bufO1, semg0, semg1, semw0, semw1):
    wid = lax.axis_index("s") * 2 + lax.axis_index("c")
    base = wid * (EPH // NW)
    ibase = h * EPH + base
    isv = (isv0, isv1)
    idv = (idv0, idv1)
    bufA = (bufA0, bufA1)
    bufB = (bufB0, bufB1)
    bufO = (bufO0, bufO1)
    semg = (semg0, semg1)
    semw = (semw0, semw1)
    KK = KH // 2

    pltpu.sync_copy(si.at[pl.ds(ibase, C)], isv0)
    pltpu.sync_copy(di.at[pl.ds(ibase, C)], idv0)
    pltpu.async_copy(srcT.at[isv0], bufA0, semg0)
    pltpu.async_copy(dstT.at[idv0], bufB0, semg0)

    def outer(kk, carry):
        for b in (0, 1):
            bp = 1 - b
            k = 2 * kk + b
            e0 = base + k * C
            i1 = ibase + (k + 1) * C

            def prefetch():
                pltpu.sync_copy(si.at[pl.ds(i1, C)], isv[bp])
                pltpu.sync_copy(di.at[pl.ds(i1, C)], idv[bp])
                pltpu.async_copy(srcT.at[isv[bp]], bufA[bp], semg[bp])
                pltpu.async_copy(dstT.at[idv[bp]], bufB[bp], semg[bp])

            if b == 0:
                prefetch()
            else:
                @pl.when(kk < KK - 1)
                def _():
                    prefetch()

            pltpu.make_async_copy(srcT.at[isv[b]], bufA[b], semg[b]).wait()
            pltpu.make_async_copy(dstT.at[idv[b]], bufB[b], semg[b]).wait()

            @pl.when(kk > 0)
            def _():
                pltpu.make_async_copy(bufO[b], out.at[pl.ds(base, C)],
                                      semw[b]).wait()

            def add_row(r, c2):
                for cc in range(R // 16):
                    sl = pl.ds(cc * 16, 16)
                    bufO[b][r, sl] = bufA[b][r, sl] + bufB[b][r, sl]
                return c2

            lax.fori_loop(0, C, add_row, 0)
            pltpu.async_copy(bufO[b], out.at[pl.ds(e0, C)], semw[b])
        return carry

    lax.fori_loop(0, KK, outer, 0)
    pltpu.make_async_copy(bufO0, out.at[pl.ds(base, C)], semw0).wait()
    pltpu.make_async_copy(bufO1, out.at[pl.ds(base, C)], semw1).wait()
  return _sc_gather_body


@functools.cache
def _sc_gather_kernel(h):
    mesh = plsc.VectorSubcoreMesh(core_axis_name="c", subcore_axis_name="s")
    return pl.kernel(
        _make_gather_body(h),
        out_type=jax.ShapeDtypeStruct((EPH, R), F32),
        mesh=mesh,
        scratch_types=[
            pltpu.VMEM((C,), jnp.int32),
            pltpu.VMEM((C,), jnp.int32),
            pltpu.VMEM((C,), jnp.int32),
            pltpu.VMEM((C,), jnp.int32),
            pltpu.VMEM((C, R), F32),
            pltpu.VMEM((C, R), F32),
            pltpu.VMEM((C, R), F32),
            pltpu.VMEM((C, R), F32),
            pltpu.VMEM((C, R), F32),
            pltpu.VMEM((C, R), F32),
            pltpu.SemaphoreType.DMA,
            pltpu.SemaphoreType.DMA,
            pltpu.SemaphoreType.DMA,
            pltpu.SemaphoreType.DMA,
        ],
    )


def _make_scatter_body(h):
  def _sc_scatter_body(rows, di, zeros, out_num, acc, idv0, idv1, buf0, buf1,
                       semr0, semr1, semsc0, semsc1):
    c = lax.axis_index("c")
    s = lax.axis_index("s")
    wid = s * 2 + c
    base = wid * (EPH // NW)
    ibase = h * EPH + base
    rz = NP // 16
    idv = (idv0, idv1)
    buf = (buf0, buf1)
    semr = (semr0, semr1)
    semsc = (semsc0, semsc1)
    KK = KH // 2

    pltpu.sync_copy(zeros.at[pl.ds(s * rz, rz)], acc.at[pl.ds(s * rz, rz)])
    plsc.subcore_barrier()

    pltpu.sync_copy(di.at[pl.ds(ibase, C)], idv0)
    pltpu.async_copy(rows.at[pl.ds(base, C)], buf0, semr0)

    def outer(kk, carry):
        for b in (0, 1):
            bp = 1 - b
            k = 2 * kk + b
            e0 = base + k * C
            e1 = e0 + C
            i1 = ibase + (k + 1) * C

            pltpu.make_async_copy(rows.at[pl.ds(e0, C)], buf[b],
                                  semr[b]).wait()
            pltpu.async_copy(buf[b], acc.at[idv[b]], semsc[b], add=True)

            def wait_prev():
                pltpu.make_async_copy(buf[bp], acc.at[idv[bp]],
                                      semsc[bp]).wait()

            def prefetch():
                pltpu.sync_copy(di.at[pl.ds(i1, C)], idv[bp])
                pltpu.async_copy(rows.at[pl.ds(e1, C)], buf[bp], semr[bp])

            if b == 0:
                @pl.when(kk > 0)
                def _():
                    wait_prev()

                prefetch()
            else:
                wait_prev()

                @pl.when(kk < KK - 1)
                def _():
                    prefetch()
        return carry

    lax.fori_loop(0, KK, outer, 0)
    pltpu.make_async_copy(buf1, acc.at[idv1], semsc1).wait()
    plsc.subcore_barrier()
    pltpu.sync_copy(acc.at[pl.ds(s * rz, rz)],
                    out_num.at[pl.ds(c * NP + s * rz, rz)])
  return _sc_scatter_body


@functools.cache
def _sc_scatter_kernel(h):
    mesh = plsc.VectorSubcoreMesh(core_axis_name="c", subcore_axis_name="s")
    return pl.kernel(
        _make_scatter_body(h),
        out_type=jax.ShapeDtypeStruct((2 * NP, R), F32),
        mesh=mesh,
        scratch_types=[
            pltpu.VMEM_SHARED((NP, R), F32),
            pltpu.VMEM((C,), jnp.int32),
            pltpu.VMEM((C,), jnp.int32),
            pltpu.VMEM((C, R), F32),
            pltpu.VMEM((C, R), F32),
            pltpu.SemaphoreType.DMA,
            pltpu.SemaphoreType.DMA,
            pltpu.SemaphoreType.DMA,
            pltpu.SemaphoreType.DMA,
        ],
    )


def kernel(node_x, edge_attr, globals_x, params, node_batch, edge_index,
           edge_graph_index):
    node_x_p = jnp.pad(node_x, ((0, NP - N), (0, 0)))
    edge_attr_p = jnp.pad(edge_attr, ((0, EP - E), (0, 0)))
    si = jnp.pad(edge_index[0].astype(jnp.int32), (0, EP - E))
    di = jnp.pad(edge_index[1].astype(jnp.int32), (0, EP - E))
    di_s = jnp.pad(edge_index[1].astype(jnp.int32), (0, EP - E),
                   constant_values=N)
    nb3 = jnp.pad(node_batch.astype(jnp.int32), (0, NP - N),
                  constant_values=G).reshape(NP // BN, 1, BN)
    eg3 = jnp.pad(edge_graph_index.astype(jnp.int32), (0, EP - E),
                  constant_values=G).reshape(EP // BE, 1, BE)
    zeros_acc = jnp.zeros((NP, R), F32)
    di3 = di_s.reshape(EP // BE, 1, BE)

    ap = params['attn']
    pe = params['edge_upd']
    w0 = pe['W0']
    wsrc, wdst, wefc = w0[0:2 * R], w0[2 * R:4 * R], w0[4 * R:6 * R]
    a2 = ap['a'][2 * R:].reshape(R, 1)

    nd0 = _mlp_rows(node_x_p, params['node_enc'], BN)
    ef0 = _mlp_rows(edge_attr_p, params['edge_enc'], BE,
                    out_dtype=jnp.bfloat16)

    nd, ef = nd0, ef0
    for r in range(3):
        srcT, dstT, g1 = _nodeA(nd0, nd, wsrc, wdst, ap['W1'])
        eargs = (di3, wefc, pe['b0'].reshape(1, R),
                 pe['W1'], pe['b1'].reshape(1, R),
                 pe['g'].reshape(1, R), pe['beta'].reshape(1, R),
                 ap['W2'], a2)
        edt = F32 if r == 2 else jnp.bfloat16
        msg0 = _sc_gather_kernel(0)(srcT, dstT, si, di)
        msg1 = _sc_gather_kernel(1)(srcT, dstT, si, di)
        ef_a = ef[0] if isinstance(ef, tuple) else ef
        ef_b = ef[1] if isinstance(ef, tuple) else ef
        loc = isinstance(ef, tuple)
        efa, rows0, den0 = _edgeB(msg0, ef0, ef_a, *eargs,
                                  ef_out_dtype=edt, h=0, ef_local=loc)
        p0 = _sc_scatter_kernel(0)(rows0, di_s, zeros_acc)
        efb, rows1, den1 = _edgeB(msg1, ef0, ef_b, *eargs,
                                  ef_out_dtype=edt, h=1, ef_local=loc)
        p1 = _sc_scatter_kernel(1)(rows1, di_s, zeros_acc)
        nd = _nodeD(p0, p1, den0, den1, g1, params['node_upd'])
        ef = (efa, efb)

    ef = jnp.concatenate([efa, efb], axis=0)
    na = _seg_agg(nb3, nd, BN)
    ea = _seg_agg(eg3, ef, BE)
    gd = _glob(globals_x, na, ea, params['glob_enc'], params['glob_upd'])
    return (nd[:N], ef[:E], gd)


# speedup vs baseline: 1.0589x; 1.0005x over previous
import functools

import jax
import jax.numpy as jnp
from jax import lax
from jax.experimental import pallas as pl
from jax.experimental.pallas import tpu as pltpu
from jax.experimental.pallas import tpu_sc as plsc

N = 10000
E = 320000
G = 64
R = 128
NP = 10240
EP = 327680
NW = 32
C = 128
K = EP // NW // C
EPH = EP // 2
KH = EPH // NW // C
DR = NP // R
BN = 1024
BE = 8192
F32 = jnp.float32


def _ln(x, g, b):
    m = jnp.mean(x, axis=-1, keepdims=True)
    v = jnp.mean((x - m) * (x - m), axis=-1, keepdims=True)
    return (x - m) * jax.lax.rsqrt(v + 1e-5) * g + b


def _dot(a, b):
    return jnp.dot(a, b, preferred_element_type=F32)


def _leaky(x):
    return jnp.where(x >= 0, x, 0.2 * x)


def _mlp_body(x_ref, w0, b0, w1, b1, g, beta, o_ref):
    h = jnp.maximum(_dot(x_ref[...], w0[...]) + b0[...], 0.0)
    o_ref[...] = _ln(_dot(h, w1[...]) + b1[...],
                     g[...], beta[...]).astype(o_ref.dtype)


def _mlp_rows(x, p, bm, out_dtype=F32):
    m, d_in = x.shape
    wspec = [
        pl.BlockSpec((d_in, R), lambda i: (0, 0)),
        pl.BlockSpec((1, R), lambda i: (0, 0)),
        pl.BlockSpec((R, R), lambda i: (0, 0)),
        pl.BlockSpec((1, R), lambda i: (0, 0)),
        pl.BlockSpec((1, R), lambda i: (0, 0)),
        pl.BlockSpec((1, R), lambda i: (0, 0)),
    ]
    return pl.pallas_call(
        _mlp_body,
        grid=(m // bm,),
        in_specs=[pl.BlockSpec((bm, d_in), lambda i: (i, 0))] + wspec,
        out_specs=pl.BlockSpec((bm, R), lambda i: (i, 0)),
        out_shape=jax.ShapeDtypeStruct((m, R), out_dtype),
    )(x, p['W0'], p['b0'].reshape(1, R), p['W1'], p['b1'].reshape(1, R),
      p['g'].reshape(1, R), p['beta'].reshape(1, R))


def _nodeA_body(nd0_ref, nd_ref, wsrc, wdst, w1a, srcT_ref, dstT_ref,
                g1_ref):
    ndc = jnp.concatenate([nd0_ref[...], nd_ref[...]], axis=1)
    srcT_ref[...] = _dot(ndc, wsrc[...])
    dstT_ref[...] = _dot(ndc, wdst[...])
    g1_ref[...] = _dot(ndc, w1a[...])


def _nodeA(nd0, nd, wsrc, wdst, w1a):
    return pl.pallas_call(
        _nodeA_body,
        grid=(NP // BN,),
        in_specs=[
            pl.BlockSpec((BN, R), lambda i: (i, 0)),
            pl.BlockSpec((BN, R), lambda i: (i, 0)),
            pl.BlockSpec((2 * R, R), lambda i: (0, 0)),
            pl.BlockSpec((2 * R, R), lambda i: (0, 0)),
            pl.BlockSpec((2 * R, 2 * R), lambda i: (0, 0)),
        ],
        out_specs=[
            pl.BlockSpec((BN, R), lambda i: (i, 0)),
            pl.BlockSpec((BN, R), lambda i: (i, 0)),
            pl.BlockSpec((BN, 2 * R), lambda i: (i, 0)),
        ],
        out_shape=[
            jax.ShapeDtypeStruct((NP, R), F32),
            jax.ShapeDtypeStruct((NP, R), F32),
            jax.ShapeDtypeStruct((NP, 2 * R), F32),
        ],
    )(nd0, nd, wsrc, wdst, w1a)


def _edgeB_body(msg_ref, ef0_ref, ef_ref, di_ref, wefc, b0, w1e, b1, g, beta,
                w2, a2, efn_ref, rows_ref, den_ref):
    @pl.when(pl.program_id(0) == 0)
    def _():
        den_ref[...] = jnp.zeros_like(den_ref)

    efc = jnp.concatenate([ef0_ref[...], ef_ref[...]], axis=1).astype(F32)
    msg = msg_ref[...].astype(F32)
    h = jnp.maximum(msg + _dot(efc, wefc[...]) + b0[...], 0.0)
    efn = _ln(_dot(h, w1e[...]) + b1[...], g[...], beta[...])
    efn_ref[...] = efn.astype(efn_ref.dtype)
    g2 = _dot(efn, w2[...])
    t2 = _dot(_leaky(g2), a2[...])
    ex = jnp.exp(t2)
    rows_ref[...] = ex * g2
    di = di_ref[0, 0, :]
    bm = di.shape[0]
    lane = lax.broadcasted_iota(jnp.int32, (bm, R), 1)
    dlocal = jnp.where(lane == (di & 127)[:, None], ex, 0.0)
    ohhi = (lax.broadcasted_iota(jnp.int32, (DR, bm), 0)
            == lax.shift_right_logical(di, 7)[None, :]).astype(F32)
    den_ref[...] += _dot(ohhi, dlocal)


def _edgeB(msg, ef0, ef, di3, wefc, b0, w1e, b1, g, beta, w2, a2,
           ef_out_dtype=F32, h=0, ef_local=False):
    hb = h * (EPH // BE)
    efmap = (lambda i: (i, 0)) if ef_local else (lambda i, hb=hb: (i + hb, 0))
    return pl.pallas_call(
        _edgeB_body,
        grid=(EPH // BE,),
        in_specs=[
            pl.BlockSpec((BE, R), lambda i: (i, 0)),
            pl.BlockSpec((BE, R), lambda i, hb=hb: (i + hb, 0)),
            pl.BlockSpec((BE, R), efmap),
            pl.BlockSpec((1, 1, BE), lambda i, hb=hb: (i + hb, 0, 0)),
            pl.BlockSpec((2 * R, R), lambda i: (0, 0)),
            pl.BlockSpec((1, R), lambda i: (0, 0)),
            pl.BlockSpec((R, R), lambda i: (0, 0)),
            pl.BlockSpec((1, R), lambda i: (0, 0)),
            pl.BlockSpec((1, R), lambda i: (0, 0)),
            pl.BlockSpec((1, R), lambda i: (0, 0)),
            pl.BlockSpec((R, R), lambda i: (0, 0)),
            pl.BlockSpec((R, 1), lambda i: (0, 0)),
        ],
        out_specs=[
            pl.BlockSpec((BE, R), lambda i: (i, 0)),
            pl.BlockSpec((BE, R), lambda i: (i, 0)),
            pl.BlockSpec((DR, R), lambda i: (0, 0)),
        ],
        out_shape=[
            jax.ShapeDtypeStruct((EPH, R), ef_out_dtype),
            jax.ShapeDtypeStruct((EPH, R), F32),
            jax.ShapeDtypeStruct((DR, R), F32),
        ],
    )(msg, ef0, ef, di3, wefc, b0, w1e, b1, g, beta, w2, a2)


def _nodeD_body(p0_ref, p1_ref, p2_ref, p3_ref, dena_ref, denb_ref, g1_ref,
                w0, b0, w1, b1, g, beta, o_ref):
    num = (p0_ref[...] + p1_ref[...]) + (p2_ref[...] + p3_ref[...])
    den = dena_ref[...] + denb_ref[...]
    rec = 1.0 / (den + 1e-16)
    recb = jnp.reshape(
        jax.lax.broadcast_in_dim(rec, (BN // R, R, R), (0, 1)), (BN, R))
    agg = num * recb
    x = jnp.concatenate([g1_ref[...], agg], axis=1)
    h = jnp.maximum(_dot(x, w0[...]) + b0[...], 0.0)
    o_ref[...] = _ln(_dot(h, w1[...]) + b1[...], g[...], beta[...])


def _nodeD(pa, pb, dena, denb, g1, p):
    nb = NP // BN
    br = BN // R
    return pl.pallas_call(
        _nodeD_body,
        grid=(nb,),
        in_specs=[
            pl.BlockSpec((BN, R), lambda i: (i, 0)),
            pl.BlockSpec((BN, R), lambda i, nb=nb: (i + nb, 0)),
            pl.BlockSpec((BN, R), lambda i: (i, 0)),
            pl.BlockSpec((BN, R), lambda i, nb=nb: (i + nb, 0)),
            pl.BlockSpec((br, R), lambda i: (i, 0)),
            pl.BlockSpec((br, R), lambda i: (i, 0)),
            pl.BlockSpec((BN, 2 * R), lambda i: (i, 0)),
            pl.BlockSpec((3 * R, R), lambda i: (0, 0)),
            pl.BlockSpec((1, R), lambda i: (0, 0)),
            pl.BlockSpec((R, R), lambda i: (0, 0)),
            pl.BlockSpec((1, R), lambda i: (0, 0)),
            pl.BlockSpec((1, R), lambda i: (0, 0)),
            pl.BlockSpec((1, R), lambda i: (0, 0)),
        ],
        out_specs=pl.BlockSpec((BN, R), lambda i: (i, 0)),
        out_shape=jax.ShapeDtypeStruct((NP, R), F32),
    )(pa, pa, pb, pb, dena, denb, g1, p['W0'], p['b0'].reshape(1, R),
      p['W1'], p['b1'].reshape(1, R), p['g'].reshape(1, R),
      p['beta'].reshape(1, R))


def _seg_body(ids_ref, x_ref, o_ref):
    @pl.when(pl.program_id(0) == 0)
    def _():
        o_ref[...] = jnp.zeros_like(o_ref)

    ids = ids_ref[0, 0, :]
    bm = ids.shape[0]
    oh = (lax.broadcasted_iota(jnp.int32, (G, bm), 0)
          == ids[None, :]).astype(F32)
    o_ref[...] += _dot(oh, x_ref[...])


def _seg_agg(ids3, x, bm):
    m = x.shape[0]
    return pl.pallas_call(
        _seg_body,
        grid=(m // bm,),
        in_specs=[
            pl.BlockSpec((1, 1, bm), lambda i: (i, 0, 0)),
            pl.BlockSpec((bm, R), lambda i: (i, 0)),
        ],
        out_specs=pl.BlockSpec((G, R), lambda i: (0, 0)),
        out_shape=jax.ShapeDtypeStruct((G, R), F32),
    )(ids3, x)


def _glob_body(gx_ref, na_ref, ea_ref, w0e, b0e, w1e, b1e, ge, be,
               w0u, b0u, w1u, b1u, gu, bu, o_ref):
    h = jnp.maximum(_dot(gx_ref[...], w0e[...]) + b0e[...], 0.0)
    gd = _ln(_dot(h, w1e[...]) + b1e[...], ge[...], be[...])
    u = jnp.concatenate([gd, na_ref[...], ea_ref[...]], axis=1)
    h2 = jnp.maximum(_dot(u, w0u[...]) + b0u[...], 0.0)
    o_ref[...] = _ln(_dot(h2, w1u[...]) + b1u[...], gu[...], bu[...])


def _glob(gx, na, ea, pe, pu):
    d_g = gx.shape[1]
    return pl.pallas_call(
        _glob_body,
        grid=(1,),
        in_specs=[
            pl.BlockSpec((G, d_g), lambda i: (0, 0)),
            pl.BlockSpec((G, R), lambda i: (0, 0)),
            pl.BlockSpec((G, R), lambda i: (0, 0)),
            pl.BlockSpec((d_g, R), lambda i: (0, 0)),
            pl.BlockSpec((1, R), lambda i: (0, 0)),
            pl.BlockSpec((R, R), lambda i: (0, 0)),
            pl.BlockSpec((1, R), lambda i: (0, 0)),
            pl.BlockSpec((1, R), lambda i: (0, 0)),
            pl.BlockSpec((1, R), lambda i: (0, 0)),
            pl.BlockSpec((3 * R, R), lambda i: (0, 0)),
            pl.BlockSpec((1, R), lambda i: (0, 0)),
            pl.BlockSpec((R, R), lambda i: (0, 0)),
            pl.BlockSpec((1, R), lambda i: (0, 0)),
            pl.BlockSpec((1, R), lambda i: (0, 0)),
            pl.BlockSpec((1, R), lambda i: (0, 0)),
        ],
        out_specs=pl.BlockSpec((G, R), lambda i: (0, 0)),
        out_shape=jax.ShapeDtypeStruct((G, R), F32),
    )(gx, na, ea, pe['W0'], pe['b0'].reshape(1, R), pe['W1'],
      pe['b1'].reshape(1, R), pe['g'].reshape(1, R), pe['beta'].reshape(1, R),
      pu['W0'], pu['b0'].reshape(1, R), pu['W1'], pu['b1'].reshape(1, R),
      pu['g'].reshape(1, R), pu['beta'].reshape(1, R))


def _make_gather_body(h):
  def _sc_gather_body(srcT, dstT, si, di, out,
                      isv0, idv0, isv1, idv1, bufA0, bufB0, bufA1, bufB1,
                      bufO0, bufO1, semg0, semg1, semw0, semw1):
    wid = lax.axis_index("s") * 2 + lax.axis_index("c")
    base = wid * (EPH // NW)
    ibase = h * EPH + base
    isv = (isv0, isv1)
    idv = (idv0, idv1)
    bufA = (bufA0, bufA1)
    bufB = (bufB0, bufB1)
    bufO = (bufO0, bufO1)
    semg = (semg0, semg1)
    semw = (semw0, semw1)
    KK = KH // 2

    pltpu.sync_copy(si.at[pl.ds(ibase, C)], isv0)
    pltpu.sync_copy(di.at[pl.ds(ibase, C)], idv0)
    pltpu.async_copy(srcT.at[isv0], bufA0, semg0)
    pltpu.async_copy(dstT.at[idv0], bufB0, semg0)

    def outer(kk, carry):
        for b in (0, 1):
            bp = 1 - b
            k = 2 * kk + b
            e0 = base + k * C
            i1 = ibase + (k + 1) * C

            def prefetch():
                pltpu.sync_copy(si.at[pl.ds(i1, C)], isv[bp])
                pltpu.sync_copy(di.at[pl.ds(i1, C)], idv[bp])
                pltpu.async_copy(srcT.at[isv[bp]], bufA[bp], semg[bp])
                pltpu.async_copy(dstT.at[idv[bp]], bufB[bp], semg[bp])

            if b == 0:
                prefetch()
            else:
                @pl.when(kk < KK - 1)
                def _():
                    prefetch()

            pltpu.make_async_copy(srcT.at[isv[b]], bufA[b], semg[b]).wait()
            pltpu.make_async_copy(dstT.at[idv[b]], bufB[b], semg[b]).wait()

            @pl.when(kk > 0)
            def _():
                pltpu.make_async_copy(bufO[b], out.at[pl.ds(base, C)],
                                      semw[b]).wait()

            def add_row(r, c2):
                for cc in range(R // 16):
                    sl = pl.ds(cc * 16, 16)
                    bufO[b][r, sl] = bufA[b][r, sl] + bufB[b][r, sl]
                return c2

            lax.fori_loop(0, C, add_row, 0)
            pltpu.async_copy(bufO[b], out.at[pl.ds(e0, C)], semw[b])
        return carry

    lax.fori_loop(0, KK, outer, 0)
    pltpu.make_async_copy(bufO0, out.at[pl.ds(base, C)], semw0).wait()
    pltpu.make_async_copy(bufO1, out.at[pl.ds(base, C)], semw1).wait()
  return _sc_gather_body


@functools.cache
def _sc_gather_kernel(h):
    mesh = plsc.VectorSubcoreMesh(core_axis_name="c", subcore_axis_name="s")
    return pl.kernel(
        _make_gather_body(h),
        out_type=jax.ShapeDtypeStruct((EPH, R), F32),
        mesh=mesh,
        scratch_types=[
            pltpu.VMEM((C,), jnp.int32),
            pltpu.VMEM((C,), jnp.int32),
            pltpu.VMEM((C,), jnp.int32),
            pltpu.VMEM((C,), jnp.int32),
            pltpu.VMEM((C, R), F32),
            pltpu.VMEM((C, R), F32),
            pltpu.VMEM((C, R), F32),
            pltpu.VMEM((C, R), F32),
            pltpu.VMEM((C, R), F32),
            pltpu.VMEM((C, R), F32),
            pltpu.SemaphoreType.DMA,
            pltpu.SemaphoreType.DMA,
            pltpu.SemaphoreType.DMA,
            pltpu.SemaphoreType.DMA,
        ],
    )


def _make_scatter_body(h):
  def _sc_scatter_body(rows, di, zeros, out_num, acc, idv0, idv1, buf0, buf1,
                       semr0, semr1, semsc0, semsc1):
    c = lax.axis_index("c")
    s = lax.axis_index("s")
    wid = s * 2 + c
    base = wid * (EPH // NW)
    ibase = h * EPH + base
    rz = NP // 16
    idv = (idv0, idv1)
    buf = (buf0, buf1)
    semr = (semr0, semr1)
    semsc = (semsc0, semsc1)
    KK = KH // 2

    pltpu.sync_copy(zeros.at[pl.ds(s * rz, rz)], acc.at[pl.ds(s * rz, rz)])
    plsc.subcore_barrier()

    pltpu.sync_copy(di.at[pl.ds(ibase, C)], idv0)
    pltpu.async_copy(rows.at[pl.ds(base, C)], buf0, semr0)

    def outer(kk, carry):
        for b in (0, 1):
            bp = 1 - b
            k = 2 * kk + b
            e0 = base + k * C
            e1 = e0 + C
            i1 = ibase + (k + 1) * C

            pltpu.make_async_copy(rows.at[pl.ds(e0, C)], buf[b],
                                  semr[b]).wait()
            pltpu.async_copy(buf[b], acc.at[idv[b]], semsc[b], add=True)

            def wait_prev():
                pltpu.make_async_copy(buf[bp], acc.at[idv[bp]],
                                      semsc[bp]).wait()

            def prefetch():
                pltpu.sync_copy(di.at[pl.ds(i1, C)], idv[bp])
                pltpu.async_copy(rows.at[pl.ds(e1, C)], buf[bp], semr[bp])

            if b == 0:
                @pl.when(kk > 0)
                def _():
                    wait_prev()

                prefetch()
            else:
                wait_prev()

                @pl.when(kk < KK - 1)
                def _():
                    prefetch()
        return carry

    lax.fori_loop(0, KK, outer, 0)
    pltpu.make_async_copy(buf1, acc.at[idv1], semsc1).wait()
    plsc.subcore_barrier()
    pltpu.sync_copy(acc.at[pl.ds(s * rz, rz)],
                    out_num.at[pl.ds(c * NP + s * rz, rz)])
  return _sc_scatter_body


@functools.cache
def _sc_scatter_kernel(h):
    mesh = plsc.VectorSubcoreMesh(core_axis_name="c", subcore_axis_name="s")
    return pl.kernel(
        _make_scatter_body(h),
        out_type=jax.ShapeDtypeStruct((2 * NP, R), F32),
        mesh=mesh,
        scratch_types=[
            pltpu.VMEM_SHARED((NP, R), F32),
            pltpu.VMEM((C,), jnp.int32),
            pltpu.VMEM((C,), jnp.int32),
            pltpu.VMEM((C, R), F32),
            pltpu.VMEM((C, R), F32),
            pltpu.SemaphoreType.DMA,
            pltpu.SemaphoreType.DMA,
            pltpu.SemaphoreType.DMA,
            pltpu.SemaphoreType.DMA,
        ],
    )


def kernel(node_x, edge_attr, globals_x, params, node_batch, edge_index,
           edge_graph_index):
    node_x_p = jnp.pad(node_x, ((0, NP - N), (0, 0)))
    edge_attr_p = jnp.pad(edge_attr, ((0, EP - E), (0, 0)))
    si = jnp.pad(edge_index[0].astype(jnp.int32), (0, EP - E))
    di = jnp.pad(edge_index[1].astype(jnp.int32), (0, EP - E))
    di_s = jnp.pad(edge_index[1].astype(jnp.int32), (0, EP - E),
                   constant_values=N)
    nb3 = jnp.pad(node_batch.astype(jnp.int32), (0, NP - N),
                  constant_values=G).reshape(NP // BN, 1, BN)
    eg3 = jnp.pad(edge_graph_index.astype(jnp.int32), (0, EP - E),
                  constant_values=G).reshape(EP // BE, 1, BE)
    zeros_acc = jnp.zeros((NP, R), F32)
    di3 = di_s.reshape(EP // BE, 1, BE)

    ap = params['attn']
    pe = params['edge_upd']
    w0 = pe['W0']
    wsrc, wdst, wefc = w0[0:2 * R], w0[2 * R:4 * R], w0[4 * R:6 * R]
    a2 = ap['a'][2 * R:].reshape(R, 1)

    nd0 = _mlp_rows(node_x_p, params['node_enc'], BN)
    ef0 = _mlp_rows(edge_attr_p, params['edge_enc'], BE,
                    out_dtype=jnp.bfloat16)

    nd, ef = nd0, ef0
    for r in range(3):
        srcT, dstT, g1 = _nodeA(nd0, nd, wsrc, wdst, ap['W1'])
        eargs = (di3, wefc, pe['b0'].reshape(1, R),
                 pe['W1'], pe['b1'].reshape(1, R),
                 pe['g'].reshape(1, R), pe['beta'].reshape(1, R),
                 ap['W2'], a2)
        edt = F32 if r == 2 else jnp.bfloat16
        msg0 = _sc_gather_kernel(0)(srcT, dstT, si, di)
        msg1 = _sc_gather_kernel(1)(srcT, dstT, si, di)
        ef_a = ef[0] if isinstance(ef, tuple) else ef
        ef_b = ef[1] if isinstance(ef, tuple) else ef
        loc = isinstance(ef, tuple)
        efa, rows0, den0 = _edgeB(msg0, ef0, ef_a, *eargs,
                                  ef_out_dtype=edt, h=0, ef_local=loc)
        p0 = _sc_scatter_kernel(0)(rows0, di_s, zeros_acc)
        efb, rows1, den1 = _edgeB(msg1, ef0, ef_b, *eargs,
                                  ef_out_dtype=edt, h=1, ef_local=loc)
        p1 = _sc_scatter_kernel(1)(rows1, di_s, zeros_acc)
        nd = _nodeD(p0, p1, den0, den1, g1, params['node_upd'])
        ef = (efa, efb)

    ef = jnp.concatenate([efa, efb], axis=0)
    na = _seg_agg(nb3, nd, BN)
    ea = _seg_agg(eg3, ef, BE)
    gd = _glob(globals_x, na, ea, params['glob_enc'], params['glob_upd'])
    return (nd[:N], ef[:E], gd)
